# baseline (speedup 1.0000x reference)
import jax
import jax.numpy as jnp
from jax.experimental import pallas as pl

NC = 1000
D = 512

def _w_kernel(x_ref, out_ref):
    out_ref[...] = jnp.zeros_like(out_ref)

def kernel(x):
    B, _ = x.shape
    CH = 4096 * NC
    flat = pl.pallas_call(
        _w_kernel,
        grid=(B * NC // CH,),
        in_specs=[pl.BlockSpec((8, D), lambda i: (0, 0))],
        out_specs=pl.BlockSpec((CH,), lambda i: (i,)),
        out_shape=jax.ShapeDtypeStruct((B * NC,), jnp.float32),
    )(x)
    return jnp.reshape(flat, (B, NC))


# speedup vs baseline: 1.1576x; 1.1576x over previous
import functools
import jax
import jax.numpy as jnp
from jax import lax
from jax.experimental import pallas as pl
from jax.experimental.pallas import tpu as pltpu
from jax.experimental.pallas import tpu_sc as plsc

NC = 1000
D = 512


def _idx_kernel(x_ref, idx_ref):
    x = x_ref[...]
    br = x.shape[0]
    f32 = jnp.float32

    pos = jax.lax.broadcasted_iota(jnp.int32, (br, D), 1).astype(f32)
    p = x * pos
    a = ((p[:, 0:128] + p[:, 128:256]) + p[:, 256:384]) + p[:, 384:512]
    at = jnp.transpose(a)
    u = at[0:8, :]
    for j in range(1, 16):
        u = u + at[8 * j:8 * j + 8, :]
    v = u[0:4, :] + u[4:8, :]
    w = v[0:2, :] + v[2:4, :]
    ht = w[0:1, :] + w[1:2, :]
    h = jnp.transpose(ht)

    ah = jnp.abs(h)
    r = ah - f32(1000.0) * jnp.floor(ah * f32(0.001))
    r = jnp.where(r == f32(1000.0), f32(0.0), r)
    r = jnp.abs(r)
    rem = jnp.where(jnp.signbit(h), -r, r)
    fixed = jnp.where(rem < f32(0.0), rem + f32(1000.0), rem)
    idx_ref[...] = jnp.squeeze(fixed.astype(jnp.int32), 1)


def _tc_indices(x):
    B, _ = x.shape
    BR = 4096
    return pl.pallas_call(
        _idx_kernel,
        grid=(B // BR,),
        in_specs=[pl.BlockSpec((BR, D), lambda i: (i, 0))],
        out_specs=pl.BlockSpec((BR,), lambda i: (i,)),
        out_shape=jax.ShapeDtypeStruct((B,), jnp.int32),
    )(x)


NW = 32
CH = 32


def _sc_writer_body(idx_hbm, t16_hbm, zin_hbm, out_hbm,
                    zbuf0, zbuf1, idxv, t16v, sem0, sem1):
    c = lax.axis_index("c")
    s = lax.axis_index("s")
    wid = s * 2 + c
    B = idx_hbm.shape[0]
    rw = B // NW
    base = wid * rw
    pltpu.sync_copy(idx_hbm.at[pl.ds(base, rw)], idxv)
    pltpu.sync_copy(t16_hbm, t16v)
    pltpu.sync_copy(zin_hbm, zbuf0)
    pltpu.sync_copy(zin_hbm, zbuf1)

    zero16 = jnp.zeros((16,), jnp.float32)
    bufs = [zbuf0, zbuf1]
    sems = [sem0, sem1]
    nch = rw // CH
    handles = [None, None]
    cols_hist = [None, None]

    def put(buf, k, val16_of):
        for g in range(CH // 16):
            colv = idxv[pl.ds(k * CH + g * 16, 16)]
            for l in range(16):
                col = colv[l]
                cb = pl.multiple_of(jnp.bitwise_and(col, jnp.int32(-16)), 16)
                buf[g * 16 + l, pl.ds(cb, 16)] = val16_of(col)
        return None

    def one16(col):
        mod = jnp.bitwise_and(col, 15)
        return t16v[mod, pl.ds(0, 16)]

    for k in range(nch):
        b = k % 2
        if handles[b] is not None:
            handles[b].wait()
            put(bufs[b], k - 2, lambda col: zero16)
        put(bufs[b], k, one16)
        handles[b] = pltpu.async_copy(
            bufs[b], out_hbm.at[pl.ds(base + k * CH, CH)], sems[b])
    for b in range(2):
        if handles[b] is not None:
            handles[b].wait()


def _sc_onehot(idx_flat, t16, zin, B):
    mesh = plsc.VectorSubcoreMesh(core_axis_name="c", subcore_axis_name="s")
    writer = functools.partial(
        pl.kernel,
        out_type=jax.ShapeDtypeStruct((B, NC), jnp.float32),
        mesh=mesh,
        scratch_types=[
            pltpu.VMEM((CH, NC), jnp.float32),
            pltpu.VMEM((CH, NC), jnp.float32),
            pltpu.VMEM((B // NW,), jnp.int32),
            pltpu.VMEM((16, 16), jnp.float32),
            pltpu.SemaphoreType.DMA,
            pltpu.SemaphoreType.DMA,
        ],
    )(_sc_writer_body)
    return writer(idx_flat, t16, zin)


def kernel(x):
    B, _ = x.shape
    idx_flat = _tc_indices(x)
    t16 = jnp.eye(16, dtype=jnp.float32)
    zin = jnp.zeros((CH, NC), jnp.float32)
    return _sc_onehot(idx_flat, t16, zin, B)


# speedup vs baseline: 1.7589x; 1.5194x over previous
import jax
import jax.numpy as jnp
import numpy as np
from jax.experimental import pallas as pl

NC = 1000
D = 512


def _classify_kernel(x_ref, out_ref):
    x = x_ref[...]
    br = x.shape[0]
    f32 = jnp.float32

    pos = jax.lax.broadcasted_iota(jnp.int32, (br, D), 1).astype(f32)
    p = x * pos

    a = ((p[:, 0:128] + p[:, 128:256]) + p[:, 256:384]) + p[:, 384:512]

    at = jnp.transpose(a)

    u = at[0:8, :]
    for j in range(1, 16):
        u = u + at[8 * j:8 * j + 8, :]

    v = u[0:4, :] + u[4:8, :]
    w = v[0:2, :] + v[2:4, :]
    ht = w[0:1, :] + w[1:2, :]
    h = jnp.transpose(ht)

    ah = jnp.abs(h)
    r = ah - f32(1000.0) * jnp.floor(ah * f32(0.001))
    r = jnp.where(r == f32(1000.0), f32(0.0), r)
    r = jnp.abs(r)
    rem = jnp.where(jnp.signbit(h), -r, r)
    fixed = jnp.where(rem < f32(0.0), rem + f32(1000.0), rem)
    idx = fixed.astype(jnp.int32)

    classes = jax.lax.broadcasted_iota(jnp.int32, (br, NC), 1)
    out_ref[...] = (classes == idx).astype(f32)


def kernel(x):
    B, _ = x.shape
    BR = 4096
    return pl.pallas_call(
        _classify_kernel,
        grid=(B // BR,),
        in_specs=[pl.BlockSpec((BR, D), lambda i: (i, 0))],
        out_specs=pl.BlockSpec((BR, NC), lambda i: (i, 0)),
        out_shape=jax.ShapeDtypeStruct((B, NC), jnp.float32),
    )(x)
